# Initial kernel scaffold; baseline (speedup 1.0000x reference)
import functools

import jax
import jax.numpy as jnp
from jax import lax
from jax.experimental import pallas as pl
from jax.experimental.pallas import tpu as pltpu
from jax.experimental.pallas import tpu_sc as plsc

N = 10000
E = 320000
D = 128
H1 = 8
C1 = 16
C2 = 128

NC = 2
NS = 16
NW = NC * NS
EPW = E // NW
KB = 80
NBLK = EPW // KB
RPS = N // NS

_f32 = jnp.float32


def _proj_body(x_ref, w_ref, vs_ref, vd_ref, xs_ref, as_ref, ad_ref):
    xb = x_ref[...]
    xs_ref[...] = jnp.dot(xb, w_ref[...], preferred_element_type=_f32)
    as_ref[...] = jnp.dot(xb, vs_ref[...], preferred_element_type=_f32)
    ad_ref[...] = jnp.dot(xb, vd_ref[...], preferred_element_type=_f32)


def _proj(x, w, vs, vd, blk=1000):
    grid = (N // blk,)
    return pl.pallas_call(
        _proj_body,
        grid=grid,
        in_specs=[
            pl.BlockSpec((blk, D), lambda i: (i, 0)),
            pl.BlockSpec((D, D), lambda i: (0, 0)),
            pl.BlockSpec((D, 16), lambda i: (0, 0)),
            pl.BlockSpec((D, 16), lambda i: (0, 0)),
        ],
        out_specs=[
            pl.BlockSpec((blk, D), lambda i: (i, 0)),
            pl.BlockSpec((blk, 16), lambda i: (i, 0)),
            pl.BlockSpec((blk, 16), lambda i: (i, 0)),
        ],
        out_shape=[
            jax.ShapeDtypeStruct((N, D), _f32),
            jax.ShapeDtypeStruct((N, 16), _f32),
            jax.ShapeDtypeStruct((N, 16), _f32),
        ],
    )(x, w, vs, vd)


def _ptab_body(mask8, as_ref, ad_ref, p_ref):
    a_s = as_ref[...]
    a_d = ad_ref[...]
    m = a_d + jnp.max(a_s, axis=0, keepdims=True)
    m = jnp.where(m >= 0, m, 0.2 * m)
    neg_m = -m
    if mask8:
        lane = lax.broadcasted_iota(jnp.int32, neg_m.shape, 1)
        neg_m = jnp.where(lane < 8, neg_m, jnp.float32(-1e30))
    p_ref[...] = jnp.concatenate([a_d, neg_m], axis=1)


def _ptab(a_s, a_d, mask8):
    return pl.pallas_call(
        functools.partial(_ptab_body, mask8),
        out_shape=jax.ShapeDtypeStruct((N, 32), _f32),
    )(a_s, a_d)


def _mid_body(acc_ref, den_ref, e1_ref, b1_ref, w2_ref, vs2_ref, vd2_ref,
              xs2_ref, as2_ref, ad2_ref):
    acc = acc_ref[0] + acc_ref[1]
    den = den_ref[0] + den_ref[1]
    den_e = jnp.dot(den, e1_ref[...], preferred_element_type=_f32)
    h = acc / (den_e + 1e-16) + b1_ref[...]
    h = jnp.maximum(h, 0.0)
    xs2_ref[...] = jnp.dot(h, w2_ref[...], preferred_element_type=_f32)
    as2_ref[...] = jnp.dot(h, vs2_ref[...], preferred_element_type=_f32)
    ad2_ref[...] = jnp.dot(h, vd2_ref[...], preferred_element_type=_f32)


def _mid(acc, den, e1, b1, w2, vs2, vd2, blk=1000):
    grid = (N // blk,)
    return pl.pallas_call(
        _mid_body,
        grid=grid,
        in_specs=[
            pl.BlockSpec((2, blk, D), lambda i: (0, i, 0)),
            pl.BlockSpec((2, blk, 16), lambda i: (0, i, 0)),
            pl.BlockSpec((16, D), lambda i: (0, 0)),
            pl.BlockSpec((1, D), lambda i: (0, 0)),
            pl.BlockSpec((D, D), lambda i: (0, 0)),
            pl.BlockSpec((D, 16), lambda i: (0, 0)),
            pl.BlockSpec((D, 16), lambda i: (0, 0)),
        ],
        out_specs=[
            pl.BlockSpec((blk, D), lambda i: (i, 0)),
            pl.BlockSpec((blk, 16), lambda i: (i, 0)),
            pl.BlockSpec((blk, 16), lambda i: (i, 0)),
        ],
        out_shape=[
            jax.ShapeDtypeStruct((N, D), _f32),
            jax.ShapeDtypeStruct((N, 16), _f32),
            jax.ShapeDtypeStruct((N, 16), _f32),
        ],
    )(acc, den, e1, b1, w2, vs2, vd2)


def _final_body(acc_ref, den_ref, e2_ref, b2_ref, out_ref):
    acc = acc_ref[0] + acc_ref[1]
    den = den_ref[0] + den_ref[1]
    den_e = jnp.dot(den, e2_ref[...], preferred_element_type=_f32)
    out_ref[...] = acc / (den_e + 1e-16) + b2_ref[...]


def _final(acc, den, e2, b2, blk=1000):
    grid = (N // blk,)
    return pl.pallas_call(
        _final_body,
        grid=grid,
        in_specs=[
            pl.BlockSpec((2, blk, D), lambda i: (0, i, 0)),
            pl.BlockSpec((2, blk, 16), lambda i: (0, i, 0)),
            pl.BlockSpec((16, D), lambda i: (0, 0)),
            pl.BlockSpec((1, D), lambda i: (0, 0)),
        ],
        out_specs=pl.BlockSpec((blk, D), lambda i: (i, 0)),
        out_shape=jax.ShapeDtypeStruct((N, D), _f32),
    )(acc, den, e2, b2)


def _edge_pass_body(src_hbm, dst_hbm, xs_hbm, as_hbm, p_hbm, z128_hbm, z16_hbm,
                    acc_out, den_out,
                    acc_sh, den_sh, sidx_v, didx_v, xs_v, as_v, p_v, ex_v,
                    sem0, sem1, sem2):
    cid = lax.axis_index("c")
    sid = lax.axis_index("s")
    wid = sid * NC + cid

    pltpu.sync_copy(z128_hbm.at[pl.ds(sid * RPS, RPS)],
                    acc_sh.at[pl.ds(sid * RPS, RPS)])
    pltpu.sync_copy(z16_hbm.at[pl.ds(sid * RPS, RPS)],
                    den_sh.at[pl.ds(sid * RPS, RPS)])
    plsc.subcore_barrier()

    shuf = lax.rem(lax.iota(jnp.int32, 16), jnp.full((16,), 8, jnp.int32))

    @pl.loop(0, NBLK)
    def _blk(b):
        off = wid * EPW + b * KB
        pltpu.sync_copy(src_hbm.at[pl.ds(off, KB)], sidx_v)
        pltpu.sync_copy(dst_hbm.at[pl.ds(off, KB)], didx_v)
        g0 = pltpu.async_copy(xs_hbm.at[sidx_v], xs_v, sem0)
        g1 = pltpu.async_copy(as_hbm.at[sidx_v], as_v, sem1)
        g2 = pltpu.async_copy(p_hbm.at[didx_v], p_v, sem2)
        g0.wait()
        g1.wait()
        g2.wait()

        @pl.loop(0, KB)
        def _edge(i):
            a_s = as_v[i, :]
            p_a = p_v[i, 0:16]
            p_m = p_v[i, 16:32]
            t = a_s + p_a
            alpha = jnp.where(t >= 0, t, 0.2 * t)
            ex = jnp.exp(alpha + p_m)
            ex_v[i, :] = ex
            spl = jnp.take(ex, shuf)
            for j in range(8):
                sl = pl.ds(j * 16, 16)
                xs_v[i, sl] = xs_v[i, sl] * spl

        pltpu.sync_copy(xs_v, acc_sh.at[didx_v], add=True)
        pltpu.sync_copy(ex_v, den_sh.at[didx_v], add=True)

    plsc.subcore_barrier()
    pltpu.sync_copy(acc_sh.at[pl.ds(sid * RPS, RPS)],
                    acc_out.at[cid, pl.ds(sid * RPS, RPS)])
    pltpu.sync_copy(den_sh.at[pl.ds(sid * RPS, RPS)],
                    den_out.at[cid, pl.ds(sid * RPS, RPS)])


def _edge_pass(src, dst, xs, a_s, p, z128, z16):
    mesh = plsc.VectorSubcoreMesh(core_axis_name="c", subcore_axis_name="s")
    f = pl.kernel(
        _edge_pass_body,
        out_type=[
            jax.ShapeDtypeStruct((NC, N, D), _f32),
            jax.ShapeDtypeStruct((NC, N, 16), _f32),
        ],
        mesh=mesh,
        scratch_types=[
            pltpu.VMEM_SHARED((N, D), _f32),
            pltpu.VMEM_SHARED((N, 16), _f32),
            pltpu.VMEM((KB,), jnp.int32),
            pltpu.VMEM((KB,), jnp.int32),
            pltpu.VMEM((KB, D), _f32),
            pltpu.VMEM((KB, 16), _f32),
            pltpu.VMEM((KB, 32), _f32),
            pltpu.VMEM((KB, 16), _f32),
            pltpu.SemaphoreType.DMA,
            pltpu.SemaphoreType.DMA,
            pltpu.SemaphoreType.DMA,
        ],
    )
    return f(src, dst, xs, a_s, p, z128, z16)


def kernel(x, edge_index, W_src1, W_dst1, att_src1, att_dst1, b1,
           W_src2, W_dst2, att_src2, att_dst2, b2):
    src = edge_index[0].astype(jnp.int32)
    dst = edge_index[1].astype(jnp.int32)

    idx = (jnp.arange(D) % H1) * C1 + (jnp.arange(D) // H1)

    a1s = (att_src1[:, :, None] * jnp.eye(H1, dtype=_f32)[:, None, :]).reshape(D, H1)
    a1d = (att_dst1[:, :, None] * jnp.eye(H1, dtype=_f32)[:, None, :]).reshape(D, H1)
    vs1 = jnp.pad(W_src1 @ a1s, ((0, 0), (0, 8)))
    vd1 = jnp.pad(W_dst1 @ a1d, ((0, 0), (0, 8)))
    w1p = W_src1[:, idx]

    w2p = W_src2[idx, :]
    v2s = jnp.tile((w2p @ att_src2[0])[:, None], (1, 16))
    v2d = jnp.tile((W_dst2[idx, :] @ att_dst2[0])[:, None], (1, 16))
    b1p = b1[idx][None, :]
    b2r = b2[None, :]

    e1 = (jnp.arange(16)[:, None] == (jnp.arange(D)[None, :] % H1)).astype(_f32)
    e2 = (jnp.arange(16)[:, None] == 0).astype(_f32) * jnp.ones((1, D), _f32)

    z128 = jnp.zeros((N, D), _f32)
    z16 = jnp.zeros((N, 16), _f32)

    xs1, as1, ad1 = _proj(x, w1p, vs1, vd1)
    p1 = _ptab(as1, ad1, mask8=True)
    acc1, den1 = _edge_pass(src, dst, xs1, as1, p1, z128, z16)

    xs2, as2, ad2 = _mid(acc1, den1, e1, b1p, w2p, v2s, v2d)
    p2 = _ptab(as2, ad2, mask8=False)
    acc2, den2 = _edge_pass(src, dst, xs2, as2, p2, z128, z16)

    return _final(acc2, den2, e2, b2r)


# speedup vs baseline: 40.0073x; 40.0073x over previous
import functools

import jax
import jax.numpy as jnp
from jax import lax
from jax.experimental import pallas as pl
from jax.experimental.pallas import tpu as pltpu
from jax.experimental.pallas import tpu_sc as plsc

N = 10000
E = 320000
D = 128
H1 = 8
C1 = 16
C2 = 128

NC = 2
NS = 16
NW = NC * NS
EPW = E // NW
KB = 80
NBLK = EPW // KB
RPS = 624
TAIL = N - RPS * NS
TOFF = RPS * NS

_f32 = jnp.float32


def _proj_body(x_ref, w_ref, vs_ref, vd_ref, xs_ref, as_ref, ad_ref):
    xb = x_ref[...]
    xs_ref[...] = jnp.dot(xb, w_ref[...], preferred_element_type=_f32)
    as_ref[...] = jnp.dot(xb, vs_ref[...], preferred_element_type=_f32)
    ad_ref[...] = jnp.dot(xb, vd_ref[...], preferred_element_type=_f32)


def _proj(x, w, vs, vd, blk=1000):
    grid = (N // blk,)
    return pl.pallas_call(
        _proj_body,
        grid=grid,
        in_specs=[
            pl.BlockSpec((blk, D), lambda i: (i, 0)),
            pl.BlockSpec((D, D), lambda i: (0, 0)),
            pl.BlockSpec((D, 16), lambda i: (0, 0)),
            pl.BlockSpec((D, 16), lambda i: (0, 0)),
        ],
        out_specs=[
            pl.BlockSpec((blk, D), lambda i: (i, 0)),
            pl.BlockSpec((blk, 16), lambda i: (i, 0)),
            pl.BlockSpec((blk, 16), lambda i: (i, 0)),
        ],
        out_shape=[
            jax.ShapeDtypeStruct((N, D), _f32),
            jax.ShapeDtypeStruct((N, 16), _f32),
            jax.ShapeDtypeStruct((N, 16), _f32),
        ],
    )(x, w, vs, vd)


def _ptab_body(mask8, as_ref, ad_ref, p_ref):
    a_s = as_ref[...]
    a_d = ad_ref[...]
    m = a_d + jnp.max(a_s, axis=0, keepdims=True)
    m = jnp.where(m >= 0, m, 0.2 * m)
    neg_m = -m
    if mask8:
        lane = lax.broadcasted_iota(jnp.int32, neg_m.shape, 1)
        neg_m = jnp.where(lane < 8, neg_m, jnp.float32(-1e30))
    p_ref[...] = jnp.concatenate([a_d, neg_m], axis=1)


def _ptab(a_s, a_d, mask8):
    return pl.pallas_call(
        functools.partial(_ptab_body, mask8),
        out_shape=jax.ShapeDtypeStruct((N, 32), _f32),
    )(a_s, a_d)


def _mid_body(acc_ref, den_ref, e1_ref, b1_ref, w2_ref, vs2_ref, vd2_ref,
              xs2_ref, as2_ref, ad2_ref):
    acc = acc_ref[0] + acc_ref[1]
    den = den_ref[0] + den_ref[1]
    den_e = jnp.dot(den, e1_ref[...], preferred_element_type=_f32)
    h = acc / (den_e + 1e-16) + b1_ref[...]
    h = jnp.maximum(h, 0.0)
    xs2_ref[...] = jnp.dot(h, w2_ref[...], preferred_element_type=_f32)
    as2_ref[...] = jnp.dot(h, vs2_ref[...], preferred_element_type=_f32)
    ad2_ref[...] = jnp.dot(h, vd2_ref[...], preferred_element_type=_f32)


def _mid(acc, den, e1, b1, w2, vs2, vd2, blk=1000):
    grid = (N // blk,)
    return pl.pallas_call(
        _mid_body,
        grid=grid,
        in_specs=[
            pl.BlockSpec((2, blk, D), lambda i: (0, i, 0)),
            pl.BlockSpec((2, blk, 16), lambda i: (0, i, 0)),
            pl.BlockSpec((16, D), lambda i: (0, 0)),
            pl.BlockSpec((1, D), lambda i: (0, 0)),
            pl.BlockSpec((D, D), lambda i: (0, 0)),
            pl.BlockSpec((D, 16), lambda i: (0, 0)),
            pl.BlockSpec((D, 16), lambda i: (0, 0)),
        ],
        out_specs=[
            pl.BlockSpec((blk, D), lambda i: (i, 0)),
            pl.BlockSpec((blk, 16), lambda i: (i, 0)),
            pl.BlockSpec((blk, 16), lambda i: (i, 0)),
        ],
        out_shape=[
            jax.ShapeDtypeStruct((N, D), _f32),
            jax.ShapeDtypeStruct((N, 16), _f32),
            jax.ShapeDtypeStruct((N, 16), _f32),
        ],
    )(acc, den, e1, b1, w2, vs2, vd2)


def _final_body(acc_ref, den_ref, e2_ref, b2_ref, out_ref):
    acc = acc_ref[0] + acc_ref[1]
    den = den_ref[0] + den_ref[1]
    den_e = jnp.dot(den, e2_ref[...], preferred_element_type=_f32)
    out_ref[...] = acc / (den_e + 1e-16) + b2_ref[...]


def _final(acc, den, e2, b2, blk=1000):
    grid = (N // blk,)
    return pl.pallas_call(
        _final_body,
        grid=grid,
        in_specs=[
            pl.BlockSpec((2, blk, D), lambda i: (0, i, 0)),
            pl.BlockSpec((2, blk, 16), lambda i: (0, i, 0)),
            pl.BlockSpec((16, D), lambda i: (0, 0)),
            pl.BlockSpec((1, D), lambda i: (0, 0)),
        ],
        out_specs=pl.BlockSpec((blk, D), lambda i: (i, 0)),
        out_shape=jax.ShapeDtypeStruct((N, D), _f32),
    )(acc, den, e2, b2)


def _edge_pass_body(src_hbm, dst_hbm, xs_hbm, as_hbm, p_hbm, z128_hbm, z16_hbm,
                    acc_out, den_out,
                    acc_sh, den_sh, sidx_v, didx_v, xs_v, as_v, p_v, ex_v,
                    sem0, sem1, sem2):
    cid = lax.axis_index("c")
    sid = lax.axis_index("s")
    wid = sid * NC + cid

    pltpu.sync_copy(z128_hbm.at[pl.ds(sid * RPS, RPS)],
                    acc_sh.at[pl.ds(sid * RPS, RPS)])
    pltpu.sync_copy(z16_hbm.at[pl.ds(sid * RPS, RPS)],
                    den_sh.at[pl.ds(sid * RPS, RPS)])

    @pl.when(sid == 0)
    def _tail_zero():
        pltpu.sync_copy(z128_hbm.at[pl.ds(TOFF, TAIL)],
                        acc_sh.at[pl.ds(TOFF, TAIL)])
        pltpu.sync_copy(z16_hbm.at[pl.ds(TOFF, TAIL)],
                        den_sh.at[pl.ds(TOFF, TAIL)])

    plsc.subcore_barrier()

    shuf = lax.rem(lax.iota(jnp.int32, 16), jnp.full((16,), 8, jnp.int32))

    @pl.loop(0, NBLK)
    def _blk(b):
        off = wid * EPW + b * KB
        pltpu.sync_copy(src_hbm.at[pl.ds(off, KB)], sidx_v)
        pltpu.sync_copy(dst_hbm.at[pl.ds(off, KB)], didx_v)
        g0 = pltpu.async_copy(xs_hbm.at[sidx_v], xs_v, sem0)
        g1 = pltpu.async_copy(as_hbm.at[sidx_v], as_v, sem1)
        g2 = pltpu.async_copy(p_hbm.at[didx_v], p_v, sem2)
        g0.wait()
        g1.wait()
        g2.wait()

        @pl.loop(0, KB)
        def _edge(i):
            a_s = as_v[i, :]
            p_a = p_v[i, 0:16]
            p_m = p_v[i, 16:32]
            t = a_s + p_a
            alpha = jnp.where(t >= 0, t, 0.2 * t)
            ex = jnp.exp(alpha + p_m)
            ex_v[i, :] = ex
            spl = jnp.take(ex, shuf)
            for j in range(8):
                sl = pl.ds(j * 16, 16)
                xs_v[i, sl] = xs_v[i, sl] * spl

        pltpu.sync_copy(xs_v, acc_sh.at[didx_v], add=True)
        pltpu.sync_copy(ex_v, den_sh.at[didx_v], add=True)

    plsc.subcore_barrier()
    pltpu.sync_copy(acc_sh.at[pl.ds(sid * RPS, RPS)],
                    acc_out.at[cid, pl.ds(sid * RPS, RPS)])
    pltpu.sync_copy(den_sh.at[pl.ds(sid * RPS, RPS)],
                    den_out.at[cid, pl.ds(sid * RPS, RPS)])

    @pl.when(sid == 0)
    def _tail_dump():
        pltpu.sync_copy(acc_sh.at[pl.ds(TOFF, TAIL)],
                        acc_out.at[cid, pl.ds(TOFF, TAIL)])
        pltpu.sync_copy(den_sh.at[pl.ds(TOFF, TAIL)],
                        den_out.at[cid, pl.ds(TOFF, TAIL)])


def _edge_pass(src, dst, xs, a_s, p, z128, z16):
    mesh = plsc.VectorSubcoreMesh(core_axis_name="c", subcore_axis_name="s")
    f = pl.kernel(
        _edge_pass_body,
        compiler_params=pltpu.CompilerParams(use_tc_tiling_on_sc=False),
        out_type=[
            jax.ShapeDtypeStruct((NC, N, D), _f32),
            jax.ShapeDtypeStruct((NC, N, 16), _f32),
        ],
        mesh=mesh,
        scratch_types=[
            pltpu.VMEM_SHARED((N, D), _f32),
            pltpu.VMEM_SHARED((N, 16), _f32),
            pltpu.VMEM((KB,), jnp.int32),
            pltpu.VMEM((KB,), jnp.int32),
            pltpu.VMEM((KB, D), _f32),
            pltpu.VMEM((KB, 16), _f32),
            pltpu.VMEM((KB, 32), _f32),
            pltpu.VMEM((KB, 16), _f32),
            pltpu.SemaphoreType.DMA,
            pltpu.SemaphoreType.DMA,
            pltpu.SemaphoreType.DMA,
        ],
    )
    return f(src, dst, xs, a_s, p, z128, z16)


def kernel(x, edge_index, W_src1, W_dst1, att_src1, att_dst1, b1,
           W_src2, W_dst2, att_src2, att_dst2, b2):
    src = edge_index[0].astype(jnp.int32)
    dst = edge_index[1].astype(jnp.int32)

    idx = (jnp.arange(D) % H1) * C1 + (jnp.arange(D) // H1)

    a1s = (att_src1[:, :, None] * jnp.eye(H1, dtype=_f32)[:, None, :]).reshape(D, H1)
    a1d = (att_dst1[:, :, None] * jnp.eye(H1, dtype=_f32)[:, None, :]).reshape(D, H1)
    vs1 = jnp.pad(W_src1 @ a1s, ((0, 0), (0, 8)))
    vd1 = jnp.pad(W_dst1 @ a1d, ((0, 0), (0, 8)))
    w1p = W_src1[:, idx]

    w2p = W_src2[idx, :]
    v2s = jnp.tile((w2p @ att_src2[0])[:, None], (1, 16))
    v2d = jnp.tile((W_dst2[idx, :] @ att_dst2[0])[:, None], (1, 16))
    b1p = b1[idx][None, :]
    b2r = b2[None, :]

    e1 = (jnp.arange(16)[:, None] == (jnp.arange(D)[None, :] % H1)).astype(_f32)
    e2 = (jnp.arange(16)[:, None] == 0).astype(_f32) * jnp.ones((1, D), _f32)

    z128 = jnp.zeros((N, D), _f32)
    z16 = jnp.zeros((N, 16), _f32)

    xs1, as1, ad1 = _proj(x, w1p, vs1, vd1)
    p1 = _ptab(as1, ad1, mask8=True)
    acc1, den1 = _edge_pass(src, dst, xs1, as1, p1, z128, z16)

    xs2, as2, ad2 = _mid(acc1, den1, e1, b1p, w2p, v2s, v2d)
    p2 = _ptab(as2, ad2, mask8=False)
    acc2, den2 = _edge_pass(src, dst, xs2, as2, p2, z128, z16)

    return _final(acc2, den2, e2, b2r)


# speedup vs baseline: 58.5758x; 1.4641x over previous
import functools

import jax
import jax.numpy as jnp
from jax import lax
from jax.experimental import pallas as pl
from jax.experimental.pallas import tpu as pltpu
from jax.experimental.pallas import tpu_sc as plsc

N = 10000
E = 320000
D = 128
G = 144
H1 = 8
C1 = 16

NC = 2
NS = 16
NW = NC * NS
EPW = E // NW
KB = 80
NBLK = EPW // KB
NBT = E // KB
RPS = 624
TAIL = N - RPS * NS
TOFF = RPS * NS

_f32 = jnp.float32


def _proj_body(x_ref, wg_ref, vd_ref, g_ref, ad_ref):
    xb = x_ref[...]
    g_ref[...] = jnp.dot(xb, wg_ref[...], preferred_element_type=_f32)
    ad_ref[...] = jnp.dot(xb, vd_ref[...], preferred_element_type=_f32)


def _proj(x, wg, vd, blk=1000):
    d = x.shape[1]
    grid = (N // blk,)
    return pl.pallas_call(
        _proj_body,
        grid=grid,
        in_specs=[
            pl.BlockSpec((blk, d), lambda i: (i, 0)),
            pl.BlockSpec((d, G), lambda i: (0, 0)),
            pl.BlockSpec((d, 16), lambda i: (0, 0)),
        ],
        out_specs=[
            pl.BlockSpec((blk, G), lambda i: (i, 0)),
            pl.BlockSpec((blk, 16), lambda i: (i, 0)),
        ],
        out_shape=[
            jax.ShapeDtypeStruct((N, G), _f32),
            jax.ShapeDtypeStruct((N, 16), _f32),
        ],
    )(x, wg, vd)


def _ptab_body(mask8, g_ref, ad_ref, p_ref):
    a_s = g_ref[:, D:G]
    a_d = ad_ref[...]
    m = a_d + jnp.max(a_s, axis=0, keepdims=True)
    m = jnp.where(m >= 0, m, 0.2 * m)
    neg_m = -m
    if mask8:
        lane = lax.broadcasted_iota(jnp.int32, neg_m.shape, 1)
        neg_m = jnp.where(lane < 8, neg_m, jnp.float32(-1e30))
    p_ref[...] = jnp.concatenate([a_d, neg_m], axis=1)


def _ptab(g, a_d, mask8):
    return pl.pallas_call(
        functools.partial(_ptab_body, mask8),
        out_shape=jax.ShapeDtypeStruct((N, 32), _f32),
    )(g, a_d)


def _mid_body(acc_ref, e1_ref, b1_ref, wg2_ref, vd2_ref, g2_ref, ad2_ref):
    accg = acc_ref[0] + acc_ref[1]
    acc = accg[:, 0:D]
    den = accg[:, D:G]
    den_e = jnp.dot(den, e1_ref[...], preferred_element_type=_f32)
    h = acc / (den_e + 1e-16) + b1_ref[...]
    h = jnp.maximum(h, 0.0)
    g2_ref[...] = jnp.dot(h, wg2_ref[...], preferred_element_type=_f32)
    ad2_ref[...] = jnp.dot(h, vd2_ref[...], preferred_element_type=_f32)


def _mid(acc, e1, b1, wg2, vd2, blk=1000):
    grid = (N // blk,)
    return pl.pallas_call(
        _mid_body,
        grid=grid,
        in_specs=[
            pl.BlockSpec((2, blk, G), lambda i: (0, i, 0)),
            pl.BlockSpec((16, D), lambda i: (0, 0)),
            pl.BlockSpec((1, D), lambda i: (0, 0)),
            pl.BlockSpec((D, G), lambda i: (0, 0)),
            pl.BlockSpec((D, 16), lambda i: (0, 0)),
        ],
        out_specs=[
            pl.BlockSpec((blk, G), lambda i: (i, 0)),
            pl.BlockSpec((blk, 16), lambda i: (i, 0)),
        ],
        out_shape=[
            jax.ShapeDtypeStruct((N, G), _f32),
            jax.ShapeDtypeStruct((N, 16), _f32),
        ],
    )(acc, e1, b1, wg2, vd2)


def _final_body(acc_ref, e2_ref, b2_ref, out_ref):
    accg = acc_ref[0] + acc_ref[1]
    acc = accg[:, 0:D]
    den = accg[:, D:G]
    den_e = jnp.dot(den, e2_ref[...], preferred_element_type=_f32)
    out_ref[...] = acc / (den_e + 1e-16) + b2_ref[...]


def _final(acc, e2, b2, blk=1000):
    grid = (N // blk,)
    return pl.pallas_call(
        _final_body,
        grid=grid,
        in_specs=[
            pl.BlockSpec((2, blk, G), lambda i: (0, i, 0)),
            pl.BlockSpec((16, D), lambda i: (0, 0)),
            pl.BlockSpec((1, D), lambda i: (0, 0)),
        ],
        out_specs=pl.BlockSpec((blk, D), lambda i: (i, 0)),
        out_shape=jax.ShapeDtypeStruct((N, D), _f32),
    )(acc, e2, b2)


def _edge_pass_body(ei_hbm, g_hbm, p_hbm, z_hbm, acc_out,
                    acc_sh, idx_v, g_v, p_v,
                    semg0, semg1, semp0, semp1):
    cid = lax.axis_index("c")
    sid = lax.axis_index("s")
    wid = sid * NC + cid
    semg = [semg0, semg1]
    semp = [semp0, semp1]

    pltpu.sync_copy(z_hbm.at[pl.ds(sid * RPS, RPS)],
                    acc_sh.at[pl.ds(sid * RPS, RPS)])

    @pl.when(sid == 0)
    def _tail_zero():
        pltpu.sync_copy(z_hbm.at[pl.ds(TOFF, TAIL)],
                        acc_sh.at[pl.ds(TOFF, TAIL)])

    plsc.subcore_barrier()

    shuf = lax.rem(lax.iota(jnp.int32, 16), jnp.full((16,), 8, jnp.int32))

    def fire(ph, b):
        pltpu.sync_copy(ei_hbm.at[wid * NBLK + b], idx_v.at[ph])
        pltpu.async_copy(g_hbm.at[idx_v.at[ph, 0]], g_v.at[ph], semg[ph])
        pltpu.async_copy(p_hbm.at[idx_v.at[ph, 1]], p_v.at[ph], semp[ph])

    def wait(ph):
        pltpu.make_async_copy(g_hbm.at[idx_v.at[ph, 0]], g_v.at[ph],
                              semg[ph]).wait()
        pltpu.make_async_copy(p_hbm.at[idx_v.at[ph, 1]], p_v.at[ph],
                              semp[ph]).wait()

    def consume(ph):
        @pl.loop(0, KB)
        def _edge(i):
            a_s = g_v[ph, i, D:G]
            p_a = p_v[ph, i, 0:16]
            p_m = p_v[ph, i, 16:32]
            t = a_s + p_a
            alpha = jnp.where(t >= 0, t, 0.2 * t)
            ex = jnp.exp(alpha + p_m)
            g_v[ph, i, D:G] = ex
            spl = jnp.take(ex, shuf)
            for j in range(8):
                sl = pl.ds(j * 16, 16)
                g_v[ph, i, sl] = g_v[ph, i, sl] * spl

        pltpu.sync_copy(g_v.at[ph], acc_sh.at[idx_v.at[ph, 1]], add=True)

    fire(0, 0)

    @pl.loop(0, NBLK - 1, step=2)
    def _blk(b):
        fire(1, b + 1)
        wait(0)
        consume(0)
        fire(0, b + 2)
        wait(1)
        consume(1)

    wait(0)
    consume(0)

    plsc.subcore_barrier()
    pltpu.sync_copy(acc_sh.at[pl.ds(sid * RPS, RPS)],
                    acc_out.at[cid, pl.ds(sid * RPS, RPS)])

    @pl.when(sid == 0)
    def _tail_dump():
        pltpu.sync_copy(acc_sh.at[pl.ds(TOFF, TAIL)],
                        acc_out.at[cid, pl.ds(TOFF, TAIL)])


def _edge_pass(ei, g, p, z):
    mesh = plsc.VectorSubcoreMesh(core_axis_name="c", subcore_axis_name="s")
    f = pl.kernel(
        _edge_pass_body,
        compiler_params=pltpu.CompilerParams(use_tc_tiling_on_sc=False),
        out_type=jax.ShapeDtypeStruct((NC, N, G), _f32),
        mesh=mesh,
        scratch_types=[
            pltpu.VMEM_SHARED((N, G), _f32),
            pltpu.VMEM((2, 2, KB), jnp.int32),
            pltpu.VMEM((2, KB, G), _f32),
            pltpu.VMEM((2, KB, 32), _f32),
            pltpu.SemaphoreType.DMA,
            pltpu.SemaphoreType.DMA,
            pltpu.SemaphoreType.DMA,
            pltpu.SemaphoreType.DMA,
        ],
    )
    return f(ei, g, p, z)


def kernel(x, edge_index, W_src1, W_dst1, att_src1, att_dst1, b1,
           W_src2, W_dst2, att_src2, att_dst2, b2):
    ei = edge_index.astype(jnp.int32).reshape(2, NBT, KB).transpose(1, 0, 2)

    idx = (jnp.arange(D) % H1) * C1 + (jnp.arange(D) // H1)

    a1s = (att_src1[:, :, None] * jnp.eye(H1, dtype=_f32)[:, None, :]).reshape(D, H1)
    a1d = (att_dst1[:, :, None] * jnp.eye(H1, dtype=_f32)[:, None, :]).reshape(D, H1)
    vs1 = jnp.pad(W_src1 @ a1s, ((0, 0), (0, 8)))
    vd1 = jnp.pad(W_dst1 @ a1d, ((0, 0), (0, 8)))
    wg1 = jnp.concatenate([W_src1[:, idx], vs1], axis=1)

    w2p = W_src2[idx, :]
    v2s = jnp.tile((w2p @ att_src2[0])[:, None], (1, 16))
    v2d = jnp.tile((W_dst2[idx, :] @ att_dst2[0])[:, None], (1, 16))
    wg2 = jnp.concatenate([w2p, v2s], axis=1)
    b1p = b1[idx][None, :]
    b2r = b2[None, :]

    e1 = (jnp.arange(16)[:, None] == (jnp.arange(D)[None, :] % H1)).astype(_f32)
    e2 = (jnp.arange(16)[:, None] == 0).astype(_f32) * jnp.ones((1, D), _f32)

    z = jnp.zeros((N, G), _f32)

    g1, ad1 = _proj(x, wg1, vd1)
    p1 = _ptab(g1, ad1, mask8=True)
    acc1 = _edge_pass(ei, g1, p1, z)

    g2, ad2 = _mid(acc1, e1, b1p, wg2, v2d)
    p2 = _ptab(g2, ad2, mask8=False)
    acc2 = _edge_pass(ei, g2, p2, z)

    return _final(acc2, e2, b2r)


# speedup vs baseline: 85.8328x; 1.4653x over previous
import functools

import jax
import jax.numpy as jnp
from jax import lax
from jax.experimental import pallas as pl
from jax.experimental.pallas import tpu as pltpu
from jax.experimental.pallas import tpu_sc as plsc

N = 10000
E = 320000
D = 128
G = 144
H1 = 8
C1 = 16

NC = 2
NS = 16
NW = NC * NS
EPW = E // NW
KB = 80
NBLK = EPW // KB
NBT = E // KB
RPS = 624
TAIL = N - RPS * NS
TOFF = RPS * NS

_f32 = jnp.float32


def _proj_body(x_ref, wg_ref, vd_ref, g_ref, ad_ref):
    xb = x_ref[...]
    g_ref[...] = jnp.dot(xb, wg_ref[...], preferred_element_type=_f32)
    ad_ref[...] = jnp.dot(xb, vd_ref[...], preferred_element_type=_f32)


def _proj(x, wg, vd, blk=1000):
    d = x.shape[1]
    grid = (N // blk,)
    return pl.pallas_call(
        _proj_body,
        grid=grid,
        in_specs=[
            pl.BlockSpec((blk, d), lambda i: (i, 0)),
            pl.BlockSpec((d, G), lambda i: (0, 0)),
            pl.BlockSpec((d, 16), lambda i: (0, 0)),
        ],
        out_specs=[
            pl.BlockSpec((blk, G), lambda i: (i, 0)),
            pl.BlockSpec((blk, 16), lambda i: (i, 0)),
        ],
        out_shape=[
            jax.ShapeDtypeStruct((N, G), _f32),
            jax.ShapeDtypeStruct((N, 16), _f32),
        ],
    )(x, wg, vd)


def _ptab_body(mask8, g_ref, ad_ref, p_ref):
    a_s = g_ref[:, D:G]
    a_d = ad_ref[...]
    m = a_d + jnp.max(a_s, axis=0, keepdims=True)
    m = jnp.where(m >= 0, m, 0.2 * m)
    neg_m = -m
    if mask8:
        lane = lax.broadcasted_iota(jnp.int32, neg_m.shape, 1)
        neg_m = jnp.where(lane < 8, neg_m, jnp.float32(-1e30))
    p_ref[...] = jnp.concatenate([a_d, neg_m], axis=1)


def _ptab(g, a_d, mask8):
    return pl.pallas_call(
        functools.partial(_ptab_body, mask8),
        out_shape=jax.ShapeDtypeStruct((N, 32), _f32),
    )(g, a_d)


def _mid_body(acc_ref, e1_ref, b1_ref, wg2_ref, vd2_ref, g2_ref, ad2_ref):
    accg = acc_ref[0] + acc_ref[1]
    acc = accg[:, 0:D]
    den = accg[:, D:G]
    den_e = jnp.dot(den, e1_ref[...], preferred_element_type=_f32)
    h = acc / (den_e + 1e-16) + b1_ref[...]
    h = jnp.maximum(h, 0.0)
    g2_ref[...] = jnp.dot(h, wg2_ref[...], preferred_element_type=_f32)
    ad2_ref[...] = jnp.dot(h, vd2_ref[...], preferred_element_type=_f32)


def _mid(acc, e1, b1, wg2, vd2, blk=1000):
    grid = (N // blk,)
    return pl.pallas_call(
        _mid_body,
        grid=grid,
        in_specs=[
            pl.BlockSpec((2, blk, G), lambda i: (0, i, 0)),
            pl.BlockSpec((16, D), lambda i: (0, 0)),
            pl.BlockSpec((1, D), lambda i: (0, 0)),
            pl.BlockSpec((D, G), lambda i: (0, 0)),
            pl.BlockSpec((D, 16), lambda i: (0, 0)),
        ],
        out_specs=[
            pl.BlockSpec((blk, G), lambda i: (i, 0)),
            pl.BlockSpec((blk, 16), lambda i: (i, 0)),
        ],
        out_shape=[
            jax.ShapeDtypeStruct((N, G), _f32),
            jax.ShapeDtypeStruct((N, 16), _f32),
        ],
    )(acc, e1, b1, wg2, vd2)


def _final_body(acc_ref, e2_ref, b2_ref, out_ref):
    accg = acc_ref[0] + acc_ref[1]
    acc = accg[:, 0:D]
    den = accg[:, D:G]
    den_e = jnp.dot(den, e2_ref[...], preferred_element_type=_f32)
    out_ref[...] = acc / (den_e + 1e-16) + b2_ref[...]


def _final(acc, e2, b2, blk=1000):
    grid = (N // blk,)
    return pl.pallas_call(
        _final_body,
        grid=grid,
        in_specs=[
            pl.BlockSpec((2, blk, G), lambda i: (0, i, 0)),
            pl.BlockSpec((16, D), lambda i: (0, 0)),
            pl.BlockSpec((1, D), lambda i: (0, 0)),
        ],
        out_specs=pl.BlockSpec((blk, D), lambda i: (i, 0)),
        out_shape=jax.ShapeDtypeStruct((N, D), _f32),
    )(acc, e2, b2)


def _edge_pass_body(ei_hbm, g_hbm, p_hbm, z_hbm, acc_out,
                    acc_sh, idx_v, g_v, p_v,
                    semg0, semg1, semp0, semp1):
    cid = lax.axis_index("c")
    sid = lax.axis_index("s")
    wid = sid * NC + cid
    semg = [semg0, semg1]
    semp = [semp0, semp1]

    pltpu.sync_copy(z_hbm.at[pl.ds(sid * RPS, RPS)],
                    acc_sh.at[pl.ds(sid * RPS, RPS)])

    @pl.when(sid == 0)
    def _tail_zero():
        pltpu.sync_copy(z_hbm.at[pl.ds(TOFF, TAIL)],
                        acc_sh.at[pl.ds(TOFF, TAIL)])

    plsc.subcore_barrier()

    shuf = lax.rem(lax.iota(jnp.int32, 16), jnp.full((16,), 8, jnp.int32))

    def fire(ph, b):
        pltpu.sync_copy(ei_hbm.at[wid * NBLK + b], idx_v.at[ph])
        pltpu.async_copy(g_hbm.at[idx_v.at[ph, 0]], g_v.at[ph], semg[ph])
        pltpu.async_copy(p_hbm.at[idx_v.at[ph, 1]], p_v.at[ph], semp[ph])

    def wait(ph):
        pltpu.make_async_copy(g_hbm.at[idx_v.at[ph, 0]], g_v.at[ph],
                              semg[ph]).wait()
        pltpu.make_async_copy(p_hbm.at[idx_v.at[ph, 1]], p_v.at[ph],
                              semp[ph]).wait()

    def consume(ph):
        @plsc.parallel_loop(0, KB, unroll=4)
        def _edge(i):
            a_s = g_v[ph, i, D:G]
            p_a = p_v[ph, i, 0:16]
            p_m = p_v[ph, i, 16:32]
            t = a_s + p_a
            alpha = jnp.where(t >= 0, t, 0.2 * t)
            ex = jnp.exp(alpha + p_m)
            g_v[ph, i, D:G] = ex
            spl = jnp.take(ex, shuf)
            for j in range(8):
                sl = pl.ds(j * 16, 16)
                g_v[ph, i, sl] = g_v[ph, i, sl] * spl

        pltpu.sync_copy(g_v.at[ph], acc_sh.at[idx_v.at[ph, 1]], add=True)

    fire(0, 0)

    @pl.loop(0, NBLK - 1, step=2)
    def _blk(b):
        fire(1, b + 1)
        wait(0)
        consume(0)
        fire(0, b + 2)
        wait(1)
        consume(1)

    wait(0)
    consume(0)

    plsc.subcore_barrier()
    pltpu.sync_copy(acc_sh.at[pl.ds(sid * RPS, RPS)],
                    acc_out.at[cid, pl.ds(sid * RPS, RPS)])

    @pl.when(sid == 0)
    def _tail_dump():
        pltpu.sync_copy(acc_sh.at[pl.ds(TOFF, TAIL)],
                        acc_out.at[cid, pl.ds(TOFF, TAIL)])


def _edge_pass(ei, g, p, z):
    mesh = plsc.VectorSubcoreMesh(core_axis_name="c", subcore_axis_name="s")
    f = pl.kernel(
        _edge_pass_body,
        compiler_params=pltpu.CompilerParams(use_tc_tiling_on_sc=False),
        out_type=jax.ShapeDtypeStruct((NC, N, G), _f32),
        mesh=mesh,
        scratch_types=[
            pltpu.VMEM_SHARED((N, G), _f32),
            pltpu.VMEM((2, 2, KB), jnp.int32),
            pltpu.VMEM((2, KB, G), _f32),
            pltpu.VMEM((2, KB, 32), _f32),
            pltpu.SemaphoreType.DMA,
            pltpu.SemaphoreType.DMA,
            pltpu.SemaphoreType.DMA,
            pltpu.SemaphoreType.DMA,
        ],
    )
    return f(ei, g, p, z)


def kernel(x, edge_index, W_src1, W_dst1, att_src1, att_dst1, b1,
           W_src2, W_dst2, att_src2, att_dst2, b2):
    ei = edge_index.astype(jnp.int32).reshape(2, NBT, KB).transpose(1, 0, 2)

    idx = (jnp.arange(D) % H1) * C1 + (jnp.arange(D) // H1)

    a1s = (att_src1[:, :, None] * jnp.eye(H1, dtype=_f32)[:, None, :]).reshape(D, H1)
    a1d = (att_dst1[:, :, None] * jnp.eye(H1, dtype=_f32)[:, None, :]).reshape(D, H1)
    vs1 = jnp.pad(W_src1 @ a1s, ((0, 0), (0, 8)))
    vd1 = jnp.pad(W_dst1 @ a1d, ((0, 0), (0, 8)))
    wg1 = jnp.concatenate([W_src1[:, idx], vs1], axis=1)

    w2p = W_src2[idx, :]
    v2s = jnp.tile((w2p @ att_src2[0])[:, None], (1, 16))
    v2d = jnp.tile((W_dst2[idx, :] @ att_dst2[0])[:, None], (1, 16))
    wg2 = jnp.concatenate([w2p, v2s], axis=1)
    b1p = b1[idx][None, :]
    b2r = b2[None, :]

    e1 = (jnp.arange(16)[:, None] == (jnp.arange(D)[None, :] % H1)).astype(_f32)
    e2 = (jnp.arange(16)[:, None] == 0).astype(_f32) * jnp.ones((1, D), _f32)

    z = jnp.zeros((N, G), _f32)

    g1, ad1 = _proj(x, wg1, vd1)
    p1 = _ptab(g1, ad1, mask8=True)
    acc1 = _edge_pass(ei, g1, p1, z)

    g2, ad2 = _mid(acc1, e1, b1p, wg2, v2d)
    p2 = _ptab(g2, ad2, mask8=False)
    acc2 = _edge_pass(ei, g2, p2, z)

    return _final(acc2, e2, b2r)
